# baseline (speedup 1.0000x reference)
import functools

import jax
import jax.numpy as jnp
from jax import lax
from jax.experimental import pallas as pl
from jax.experimental.pallas import tpu as pltpu
from jax.experimental.pallas import tpu_sc as plsc

_N_RANKS = 20
_BATCH = 16384
_LOG2_BATCH = 14
_TOTAL = _BATCH * _N_RANKS
_NC, _NS, _L = 2, 16, 16
_NW = _NC * _NS
_BPW = _TOTAL // _NW
_S = 4
_SUB = _BPW // _S
_SUBV = _SUB // _L
_U = 8

_mesh = plsc.VectorSubcoreMesh(core_axis_name="c", subcore_axis_name="s")


@functools.partial(
    pl.kernel,
    mesh=_mesh,
    out_type=jax.ShapeDtypeStruct((_TOTAL,), jnp.float32),
    scratch_types=[
        pltpu.VMEM((_BPW,), jnp.int32),
        pltpu.VMEM((_BPW,), jnp.float32),
        pltpu.VMEM((32,), jnp.float32),
        pltpu.VMEM((7816,), jnp.float32),
        pltpu.VMEM((7816,), jnp.float32),
        pltpu.VMEM_SHARED((1000448,), jnp.float32),
        pltpu.SemaphoreType.DMA,
        pltpu.SemaphoreType.DMA,
        pltpu.SemaphoreType.DMA,
        pltpu.SemaphoreType.DMA,
    ],
)
def _pbm_kernel(x_hbm, exam_hbm, rel_hbm, out_hbm,
                idx_v, rows_v, exam_v, tbufa, tbufb, tab_s, tsem, hsem, gsem, osem):
    wid = lax.axis_index("s") * _NC + lax.axis_index("c")
    base = wid * _BPW
    sid = lax.axis_index("s")
    _TCH = 1000448 // _NS
    _HOP = _TCH // 8

    pltpu.sync_copy(x_hbm.at[pl.ds(base, _BPW)], idx_v)
    gathers = [
        pltpu.async_copy(
            rel_hbm.at[idx_v.at[pl.ds(0, _SUB)]],
            rows_v.at[pl.ds(0, _SUB)],
            hsem,
        )
    ]

    tbase = sid * _TCH
    bufs = [tbufa, tbufb]
    loads = [
        pltpu.async_copy(
            rel_hbm.at[pl.ds(tbase + k * _HOP, _HOP)], bufs[k % 2], tsem)
        for k in range(2)
    ]
    for k in range(8):
        loads[k].wait()
        pltpu.sync_copy(bufs[k % 2], tab_s.at[pl.ds(tbase + k * _HOP, _HOP)])
        if k + 2 < 8:
            loads.append(pltpu.async_copy(
                rel_hbm.at[pl.ds(tbase + (k + 2) * _HOP, _HOP)],
                bufs[k % 2], tsem))
    plsc.subcore_barrier()
    gathers += [
        pltpu.async_copy(
            tab_s.at[idx_v.at[pl.ds(j * _SUB, _SUB)]],
            rows_v.at[pl.ds(j * _SUB, _SUB)],
            gsem,
        )
        for j in range(1, _S)
    ]

    pltpu.sync_copy(exam_hbm, exam_v)
    sig0 = 1.0 / (1.0 + jnp.exp(-exam_v[pl.ds(0, _L)]))
    sig1 = 1.0 / (1.0 + jnp.exp(-exam_v[pl.ds(_L, _L)]))

    def exam_splat(c):
        i0 = jnp.full((_L,), jnp.minimum(c, _L - 1), jnp.int32)
        i1 = jnp.full((_L,), jnp.clip(c - _L, 0, _L - 1), jnp.int32)
        g0 = sig0.at[i0].get(mode="promise_in_bounds")
        g1 = sig1.at[i1].get(mode="promise_in_bounds")
        return jnp.where(c < _L, g0, g1)

    c0 = base >> _LOG2_BATCH
    e0 = exam_splat(c0)
    e1 = exam_splat(c0 + 1)
    split = jnp.minimum(((c0 + 1) << _LOG2_BATCH) - base, _BPW) // _L

    outs = []
    for j in range(_S):
        gathers[j].wait()

        def body(k, carry, j=j):
            for u in range(_U):
                i = j * _SUBV + k * _U + u
                e = jnp.where(i < split, e0, e1)
                o = i * _L
                v = rows_v[pl.ds(o, _L)]
                rows_v[pl.ds(o, _L)] = e / (1.0 + jnp.exp(-v))
            return carry

        lax.fori_loop(0, _SUBV // _U, body, 0)
        outs.append(
            pltpu.async_copy(
                rows_v.at[pl.ds(j * _SUB, _SUB)],
                out_hbm.at[pl.ds(base + j * _SUB, _SUB)],
                osem,
            )
        )
    for c in outs:
        c.wait()


def kernel(x, exam_table, rel_table):
    x_cm = x.T.reshape(-1).astype(jnp.int32)
    exam_pad = jnp.pad(exam_table.reshape(-1), (0, 32 - _N_RANKS))
    rel_wide = jnp.concatenate(
        [rel_table.T, jnp.zeros((1, 448), jnp.float32)], axis=1)
    rel_flat = rel_wide.reshape(-1)
    out_cm = _pbm_kernel(x_cm, exam_pad, rel_flat)
    return out_cm.reshape(_N_RANKS, _BATCH).T


# speedup vs baseline: 1.0257x; 1.0257x over previous
import functools

import jax
import jax.numpy as jnp
from jax import lax
from jax.experimental import pallas as pl
from jax.experimental.pallas import tpu as pltpu
from jax.experimental.pallas import tpu_sc as plsc

_N_RANKS = 20
_BATCH = 16384
_LOG2_BATCH = 14
_TOTAL = _BATCH * _N_RANKS
_NC, _NS, _L = 2, 16, 16
_NW = _NC * _NS
_BPW = _TOTAL // _NW
_S = 4
_SUB = _BPW // _S
_SUBV = _SUB // _L
_U = 8

_mesh = plsc.VectorSubcoreMesh(core_axis_name="c", subcore_axis_name="s")


@functools.partial(
    pl.kernel,
    mesh=_mesh,
    out_type=jax.ShapeDtypeStruct((_TOTAL,), jnp.float32),
    scratch_types=[
        pltpu.VMEM((_BPW,), jnp.int32),
        pltpu.VMEM((_BPW,), jnp.float32),
        pltpu.VMEM((32,), jnp.float32),
        pltpu.VMEM((7816,), jnp.float32),
        pltpu.VMEM((7816,), jnp.float32),
        pltpu.VMEM_SHARED((1000448,), jnp.float32),
        pltpu.SemaphoreType.DMA,
        pltpu.SemaphoreType.DMA,
        pltpu.SemaphoreType.DMA,
    ],
)
def _pbm_kernel(x_hbm, exam_hbm, rel_hbm, out_hbm,
                idx_v, rows_v, exam_v, tbufa, tbufb, tab_s, tsem, gsem, osem):
    wid = lax.axis_index("s") * _NC + lax.axis_index("c")
    base = wid * _BPW
    sid = lax.axis_index("s")
    _TCH = 1000448 // _NS
    _HOP = _TCH // 8

    tbase = sid * _TCH
    bufs = [tbufa, tbufb]
    loads = [
        pltpu.async_copy(
            rel_hbm.at[pl.ds(tbase + k * _HOP, _HOP)], bufs[k % 2], tsem)
        for k in range(2)
    ]
    for k in range(8):
        loads[k].wait()
        pltpu.sync_copy(bufs[k % 2], tab_s.at[pl.ds(tbase + k * _HOP, _HOP)])
        if k + 2 < 8:
            loads.append(pltpu.async_copy(
                rel_hbm.at[pl.ds(tbase + (k + 2) * _HOP, _HOP)],
                bufs[k % 2], tsem))
    pltpu.sync_copy(x_hbm.at[pl.ds(base, _BPW)], idx_v)
    plsc.subcore_barrier()
    gathers = [
        pltpu.async_copy(
            tab_s.at[idx_v.at[pl.ds(j * _SUB, _SUB)]],
            rows_v.at[pl.ds(j * _SUB, _SUB)],
            gsem,
        )
        for j in range(_S)
    ]

    pltpu.sync_copy(exam_hbm, exam_v)
    sig0 = 1.0 / (1.0 + jnp.exp(-exam_v[pl.ds(0, _L)]))
    sig1 = 1.0 / (1.0 + jnp.exp(-exam_v[pl.ds(_L, _L)]))

    def exam_splat(c):
        i0 = jnp.full((_L,), jnp.minimum(c, _L - 1), jnp.int32)
        i1 = jnp.full((_L,), jnp.clip(c - _L, 0, _L - 1), jnp.int32)
        g0 = sig0.at[i0].get(mode="promise_in_bounds")
        g1 = sig1.at[i1].get(mode="promise_in_bounds")
        return jnp.where(c < _L, g0, g1)

    c0 = base >> _LOG2_BATCH
    e0 = exam_splat(c0)
    e1 = exam_splat(c0 + 1)
    split = jnp.minimum(((c0 + 1) << _LOG2_BATCH) - base, _BPW) // _L

    outs = []
    for j in range(_S):
        gathers[j].wait()

        def body(k, carry, j=j):
            for u in range(_U):
                i = j * _SUBV + k * _U + u
                e = jnp.where(i < split, e0, e1)
                o = i * _L
                v = rows_v[pl.ds(o, _L)]
                rows_v[pl.ds(o, _L)] = e / (1.0 + jnp.exp(-v))
            return carry

        lax.fori_loop(0, _SUBV // _U, body, 0)
        outs.append(
            pltpu.async_copy(
                rows_v.at[pl.ds(j * _SUB, _SUB)],
                out_hbm.at[pl.ds(base + j * _SUB, _SUB)],
                osem,
            )
        )
    for c in outs:
        c.wait()


def kernel(x, exam_table, rel_table):
    x_cm = x.T.reshape(-1).astype(jnp.int32)
    exam_pad = jnp.pad(exam_table.reshape(-1), (0, 32 - _N_RANKS))
    rel_wide = jnp.concatenate(
        [rel_table.T, jnp.zeros((1, 448), jnp.float32)], axis=1)
    rel_flat = rel_wide.reshape(-1)
    out_cm = _pbm_kernel(x_cm, exam_pad, rel_flat)
    return out_cm.reshape(_N_RANKS, _BATCH).T
